# baseline (device time: 803518 ns/iter reference)
import jax
import jax.numpy as jnp
from jax import lax
from jax.experimental import pallas as pl
from jax.experimental.pallas import tpu as pltpu

N_DEV = 32
LOG2_N = 5


def kernel(x, w_mat):
    m_glob, k_loc = x.shape
    _, n = w_mat.shape
    m_chunk = m_glob // N_DEV

    def body(x_ref, w_ref, out_ref, w_bf, send_buf, comm_buf,
             send_sems, recv_sems, credit_sem,
             ax_send, ax_recv, ax_send_sems, ax_recv_sems):
        my = lax.axis_index("i")
        left = (my - 1) % N_DEV
        right = (my + 1) % N_DEV

        barrier = pltpu.get_barrier_semaphore()
        for nbr in (left, right):
            pl.semaphore_signal(barrier, inc=1, device_id=(nbr,),
                                device_id_type=pl.DeviceIdType.MESH)
        pl.semaphore_wait(barrier, 2)

        w_bf[...] = w_ref[...].astype(jnp.bfloat16)

        def partial_chunk(c):
            rows = x_ref[pl.ds(c * m_chunk, m_chunk), :].astype(jnp.bfloat16)
            return jnp.dot(rows, w_bf[...],
                           preferred_element_type=jnp.float32)

        p = partial_chunk((my - 1) % N_DEV)
        send_buf[0] = p.astype(jnp.bfloat16)

        s_final = None
        for s in range(N_DEV - 1):
            slot = s % 2
            if s >= 2:
                pl.semaphore_wait(credit_sem, 1)
            rdma = pltpu.make_async_remote_copy(
                src_ref=send_buf.at[slot],
                dst_ref=comm_buf.at[slot],
                send_sem=send_sems.at[slot],
                recv_sem=recv_sems.at[slot],
                device_id=(right,),
                device_id_type=pl.DeviceIdType.MESH,
            )
            rdma.start()
            p = partial_chunk((my - 2 - s) % N_DEV)
            rdma.wait()
            acc = p + comm_buf[slot].astype(jnp.float32)
            if s < N_DEV - 2:
                send_buf[(s + 1) % 2] = acc.astype(jnp.bfloat16)
            else:
                s_final = acc
            if s <= N_DEV - 4:
                pl.semaphore_signal(credit_sem, inc=1, device_id=(left,),
                                    device_id_type=pl.DeviceIdType.MESH)

        m_val = jnp.max(jnp.abs(s_final))
        for k in range(LOG2_N):
            partner = my ^ (1 << k)
            ax_send[k] = jnp.full((8, 128), m_val, dtype=jnp.float32)
            rdma = pltpu.make_async_remote_copy(
                src_ref=ax_send.at[k],
                dst_ref=ax_recv.at[k],
                send_sem=ax_send_sems.at[k],
                recv_sem=ax_recv_sems.at[k],
                device_id=(partner,),
                device_id_type=pl.DeviceIdType.MESH,
            )
            rdma.start()
            rdma.wait()
            m_val = jnp.maximum(m_val, jnp.max(ax_recv[k]))

        scale = m_val / 127.0
        q = jnp.clip(jnp.round(s_final / scale), -127.0, 127.0)
        out_ref[...] = q * scale

    return pl.pallas_call(
        body,
        out_shape=jax.ShapeDtypeStruct((m_chunk, n), jnp.float32),
        in_specs=[pl.BlockSpec(memory_space=pltpu.VMEM),
                  pl.BlockSpec(memory_space=pltpu.VMEM)],
        out_specs=pl.BlockSpec(memory_space=pltpu.VMEM),
        scratch_shapes=[
            pltpu.VMEM((k_loc, n), jnp.bfloat16),
            pltpu.VMEM((2, m_chunk, n), jnp.bfloat16),
            pltpu.VMEM((2, m_chunk, n), jnp.bfloat16),
            pltpu.SemaphoreType.DMA((2,)),
            pltpu.SemaphoreType.DMA((2,)),
            pltpu.SemaphoreType.REGULAR,
            pltpu.VMEM((LOG2_N, 8, 128), jnp.float32),
            pltpu.VMEM((LOG2_N, 8, 128), jnp.float32),
            pltpu.SemaphoreType.DMA((LOG2_N,)),
            pltpu.SemaphoreType.DMA((LOG2_N,)),
        ],
        compiler_params=pltpu.CompilerParams(collective_id=0),
    )(x, w_mat)
